# baseline (speedup 1.0000x reference)
import functools
import random

import jax
import jax.numpy as jnp
import numpy as np
from jax import lax
from jax.experimental import pallas as pl
from jax.experimental.pallas import tpu as pltpu
from jax.experimental.pallas import tpu_sc as plsc

_NUM_CORES = 2
_NUM_SUBCORES = 16
_NUM_WORKERS = _NUM_CORES * _NUM_SUBCORES
_NBUF = 2
_PIECE = 112


def _distortion_indices(h, w, a_max=3, r_max=0.7):
    random.seed(0)
    cols = h
    rows = w
    center_rows = int(np.round(random.uniform(1, rows - 2)))
    center_cols = int(np.round(random.uniform(1, cols - 2)))
    radius = random.uniform(0.03 * max(rows, cols), r_max * max(rows, cols))
    choice = random.randint(0, 1)
    spect_ratio1 = 1
    spect_ratio2 = 1
    if choice == 1:
        spect_ratio1 = random.uniform(1, a_max)
    else:
        spect_ratio2 = random.uniform(1, a_max)
    cols_np = np.arange(cols)
    rows_np = np.arange(rows)
    cols_np_t = np.tile(cols_np, (rows, 1))
    cols_pow = np.power(cols_np_t - center_cols, 2)
    rows_np_t = np.tile(rows_np, (cols, 1))
    rows_pow = np.power(rows_np_t - center_rows, 2)
    dis = np.sqrt(cols_pow + rows_pow.transpose())
    judge = (spect_ratio1 * np.abs(rows_np_t - center_rows).transpose()
             + spect_ratio2 * np.abs(cols_np_t - center_cols))
    index = np.where(judge <= radius)
    index_rows = np.rint(index[0]).astype('int64')
    index_cols = np.rint(index[1]).astype('int64')
    dis_val = dis[index]
    old_i = np.floor(dis_val / radius * (index_rows - center_rows)
                     + center_rows).astype('int64')
    old_j = np.floor(dis_val / radius * (index_cols - center_cols)
                     + center_cols).astype('int64')
    return index_rows, index_cols, old_i, old_j


@functools.lru_cache(maxsize=None)
def _row_map(b, h, w):
    ir, ic, oi, oj = _distortion_indices(h, w)
    m = np.arange(b * h * w, dtype=np.int32)
    for bb in range(b):
        off = bb * h * w
        m[off + ir * w + ic] = (off + oi * w + oj).astype(np.int32)
    return m


def _sc_row_gather(rows, rmap):
    nrows, ch = rows.shape
    per_w = nrows // _NUM_WORKERS
    assert nrows % _NUM_WORKERS == 0 and per_w % _PIECE == 0
    npp = per_w // _PIECE
    idx3 = rmap.reshape(_NUM_WORKERS, npp, _PIECE)

    mesh = plsc.VectorSubcoreMesh(
        core_axis_name="c", subcore_axis_name="s",
        num_cores=_NUM_CORES, num_subcores=_NUM_SUBCORES)

    @functools.partial(
        pl.kernel,
        out_type=jax.ShapeDtypeStruct((nrows, ch), jnp.float32),
        mesh=mesh,
        scratch_types=(
            [pltpu.VMEM((_PIECE, ch), jnp.float32)] * _NBUF + [
            pltpu.VMEM((npp, _PIECE), jnp.int32),
            pltpu.SemaphoreType.DMA((_NBUF,)),
            pltpu.SemaphoreType.DMA((_NBUF,)),
        ]),
        compiler_params=pltpu.CompilerParams(needs_layout_passes=False),
    )
    def body(feat, idxh, out, buf0, buf1, idxv,
             in_sems, out_sems):
        bufs = [buf0, buf1]
        wid = lax.axis_index("s") * _NUM_CORES + lax.axis_index("c")
        pltpu.sync_copy(idxh.at[wid], idxv)
        row0 = wid * per_w

        def issue_in(j):
            s = j % _NBUF
            pltpu.async_copy(feat.at[idxv.at[j]], bufs[s], in_sems.at[s])

        def wait_in(j):
            s = j % _NBUF
            pltpu.make_async_copy(feat.at[idxv.at[j]], bufs[s],
                                  in_sems.at[s]).wait()

        def issue_out(j):
            s = j % _NBUF
            pltpu.async_copy(bufs[s], out.at[pl.ds(row0 + j * _PIECE, _PIECE)],
                             out_sems.at[s])

        def wait_out(j):
            s = j % _NBUF
            pltpu.make_async_copy(bufs[s],
                                  out.at[pl.ds(row0 + j * _PIECE, _PIECE)],
                                  out_sems.at[s]).wait()

        for j in range(min(_NBUF - 1, npp)):
            issue_in(j)

        for j in range(npp):
            wait_in(j)
            issue_out(j)
            nxt = j + _NBUF - 1
            if nxt < npp:
                if nxt >= _NBUF:
                    wait_out(nxt - _NBUF)
                issue_in(nxt)

        for j in range(max(0, npp - _NBUF), npp):
            wait_out(j)

    return body(rows, jnp.asarray(idx3))


def kernel(feature):
    b, c, h, w = feature.shape
    rmap = _row_map(b, h, w)
    rows = feature.transpose(0, 2, 3, 1).reshape(b * h * w, c)
    out = _sc_row_gather(rows, rmap)
    return out.reshape(b, h, w, c).transpose(0, 3, 1, 2)


# speedup vs baseline: 1.0334x; 1.0334x over previous
import functools
import random

import jax
import jax.numpy as jnp
import numpy as np
from jax import lax
from jax.experimental import pallas as pl
from jax.experimental.pallas import tpu as pltpu
from jax.experimental.pallas import tpu_sc as plsc

_NUM_CORES = 2
_NUM_SUBCORES = 16
_NUM_WORKERS = _NUM_CORES * _NUM_SUBCORES
_NBUF = 5
_PIECE = 64


def _distortion_indices(h, w, a_max=3, r_max=0.7):
    random.seed(0)
    cols = h
    rows = w
    center_rows = int(np.round(random.uniform(1, rows - 2)))
    center_cols = int(np.round(random.uniform(1, cols - 2)))
    radius = random.uniform(0.03 * max(rows, cols), r_max * max(rows, cols))
    choice = random.randint(0, 1)
    spect_ratio1 = 1
    spect_ratio2 = 1
    if choice == 1:
        spect_ratio1 = random.uniform(1, a_max)
    else:
        spect_ratio2 = random.uniform(1, a_max)
    cols_np = np.arange(cols)
    rows_np = np.arange(rows)
    cols_np_t = np.tile(cols_np, (rows, 1))
    cols_pow = np.power(cols_np_t - center_cols, 2)
    rows_np_t = np.tile(rows_np, (cols, 1))
    rows_pow = np.power(rows_np_t - center_rows, 2)
    dis = np.sqrt(cols_pow + rows_pow.transpose())
    judge = (spect_ratio1 * np.abs(rows_np_t - center_rows).transpose()
             + spect_ratio2 * np.abs(cols_np_t - center_cols))
    index = np.where(judge <= radius)
    index_rows = np.rint(index[0]).astype('int64')
    index_cols = np.rint(index[1]).astype('int64')
    dis_val = dis[index]
    old_i = np.floor(dis_val / radius * (index_rows - center_rows)
                     + center_rows).astype('int64')
    old_j = np.floor(dis_val / radius * (index_cols - center_cols)
                     + center_cols).astype('int64')
    return index_rows, index_cols, old_i, old_j


@functools.lru_cache(maxsize=None)
def _row_map(b, h, w):
    ir, ic, oi, oj = _distortion_indices(h, w)
    m = np.arange(b * h * w, dtype=np.int32)
    for bb in range(b):
        off = bb * h * w
        m[off + ir * w + ic] = (off + oi * w + oj).astype(np.int32)
    return m


def _sc_row_gather(rows, rmap):
    nrows, ch = rows.shape
    per_w = nrows // _NUM_WORKERS
    assert nrows % _NUM_WORKERS == 0 and per_w % _PIECE == 0
    npp = per_w // _PIECE
    idx3 = rmap.reshape(_NUM_WORKERS, npp, _PIECE)

    mesh = plsc.VectorSubcoreMesh(
        core_axis_name="c", subcore_axis_name="s",
        num_cores=_NUM_CORES, num_subcores=_NUM_SUBCORES)

    @functools.partial(
        pl.kernel,
        out_type=jax.ShapeDtypeStruct((nrows, ch), jnp.float32),
        mesh=mesh,
        scratch_types=(
            [pltpu.VMEM((_PIECE, ch), jnp.float32)] * _NBUF + [
            pltpu.VMEM((npp, _PIECE), jnp.int32),
            pltpu.SemaphoreType.DMA((_NBUF,)),
            pltpu.SemaphoreType.DMA((_NBUF,)),
        ]),
        compiler_params=pltpu.CompilerParams(needs_layout_passes=False),
    )
    def body(feat, idxh, out, buf0, buf1, buf2, buf3, buf4, idxv,
             in_sems, out_sems):
        bufs = [buf0, buf1, buf2, buf3, buf4]
        wid = lax.axis_index("s") * _NUM_CORES + lax.axis_index("c")
        pltpu.sync_copy(idxh.at[wid], idxv)
        row0 = wid * per_w

        def issue_in(j):
            s = j % _NBUF
            pltpu.async_copy(feat.at[idxv.at[j]], bufs[s], in_sems.at[s])

        def wait_in(j):
            s = j % _NBUF
            pltpu.make_async_copy(feat.at[idxv.at[j]], bufs[s],
                                  in_sems.at[s]).wait()

        def issue_out(j):
            s = j % _NBUF
            pltpu.async_copy(bufs[s], out.at[pl.ds(row0 + j * _PIECE, _PIECE)],
                             out_sems.at[s])

        def wait_out(j):
            s = j % _NBUF
            pltpu.make_async_copy(bufs[s],
                                  out.at[pl.ds(row0 + j * _PIECE, _PIECE)],
                                  out_sems.at[s]).wait()

        for j in range(min(_NBUF - 1, npp)):
            issue_in(j)

        for j in range(npp):
            wait_in(j)
            issue_out(j)
            nxt = j + _NBUF - 1
            if nxt < npp:
                if nxt >= _NBUF:
                    wait_out(nxt - _NBUF)
                issue_in(nxt)

        for j in range(max(0, npp - _NBUF), npp):
            wait_out(j)

    return body(rows, jnp.asarray(idx3))


def kernel(feature):
    b, c, h, w = feature.shape
    rmap = _row_map(b, h, w)
    rows = feature.transpose(0, 2, 3, 1).reshape(b * h * w, c)
    out = _sc_row_gather(rows, rmap)
    return out.reshape(b, h, w, c).transpose(0, 3, 1, 2)
